# Initial kernel scaffold; baseline (speedup 1.0000x reference)
import jax
import jax.numpy as jnp
from jax.experimental import pallas as pl


def kernel(x, emb_weight, fc_weight, bias):
    raise NotImplementedError("write your pallas kernel here")


# speedup vs baseline: 1.5312x; 1.5312x over previous
import jax
import jax.numpy as jnp
from jax import lax
from jax.experimental import pallas as pl
from jax.experimental.pallas import tpu as pltpu
from jax.experimental.pallas import tpu_sc as plsc

BATCH = 4096
NCAT = 26
NCVL = 13
EMB_D = 64
LANES = 16

NCORES = 2
NSUB = 16
NWORK = NCORES * NSUB
EPW = BATCH // NWORK
CHUNK = 16
NCHUNK = EPW // CHUNK
SUBG = 4
EPS = CHUNK // SUBG
RPS = EPS * NCAT


def _fm_body(idxg, idxf, cont, emb, fc, bias, out,
             idx_v, idxf_v, cont_v, fcv, rows0, rows1, out_v, bias_v,
             sem_fc, sem0, sem1):
  wid = lax.axis_index("s") * NCORES + lax.axis_index("c")

  pltpu.sync_copy(idxg.at[pl.ds(wid * (NCHUNK * SUBG), NCHUNK * SUBG)], idx_v)
  pltpu.sync_copy(idxf.at[wid], idxf_v)
  pltpu.sync_copy(cont.at[wid], cont_v)
  pltpu.sync_copy(bias, bias_v)

  fc_descs = [
      pltpu.async_copy(fc.at[idxf_v.at[f]], fcv.at[f], sem_fc)
      for f in range(NCAT)
  ]

  for t in range(2):
    rows_b, sem_b = (rows0, sem0) if t == 0 else (rows1, sem1)
    for j in range(SUBG):
      pltpu.async_copy(emb.at[idx_v.at[t * SUBG + j]],
                       rows_b.at[pl.ds(j * RPS, RPS)], sem_b)

  for d in fc_descs:
    d.wait()
  bvec = bias_v[...]
  for c in range(NCHUNK):
    acc = bvec
    for f in range(NCAT):
      v = fcv[f, pl.ds(c * CHUNK, CHUNK)]
      if f >= NCAT - NCVL:
        v = v * cont_v[f - (NCAT - NCVL), pl.ds(c * CHUNK, CHUNK)]
      acc = acc + v
    out_v[c, :] = acc

  lane = lax.broadcasted_iota(jnp.int32, (LANES,), 0)

  def chunk_compute(t, rows_b):
    def elem(i, acc):
      base = i * NCAT
      zero = jnp.zeros((LANES,), jnp.float32)
      s0 = s1 = s2 = s3 = q = zero
      for f in range(NCAT):
        r = base + f
        e0 = rows_b[r, pl.ds(0, 16)]
        e1 = rows_b[r, pl.ds(16, 16)]
        e2 = rows_b[r, pl.ds(32, 16)]
        e3 = rows_b[r, pl.ds(48, 16)]
        s0 = s0 + e0
        s1 = s1 + e1
        s2 = s2 + e2
        s3 = s3 + e3
        q = q + (e0 * e0 + e1 * e1 + e2 * e2 + e3 * e3)
      ssq = s0 * s0 + s1 * s1 + s2 * s2 + s3 * s3
      val = 0.5 * (jnp.sum(ssq) - jnp.sum(q))
      return acc + jnp.where(lane == i, val, 0.0)

    inter = lax.fori_loop(0, CHUNK, elem, jnp.zeros((LANES,), jnp.float32))
    z = out_v[t, :] + inter
    out_v[t, :] = 1.0 / (1.0 + jnp.exp(-z))

  def drain(t, rows_b, sem_b):
    for j in range(SUBG):
      pltpu.make_async_copy(emb.at[idx_v.at[t * SUBG + j]],
                            rows_b.at[pl.ds(j * RPS, RPS)], sem_b).wait()

  def step(g, carry):
    for b in range(2):
      t = g * 2 + b
      rows_b, sem_b = (rows0, sem0) if b == 0 else (rows1, sem1)
      drain(t, rows_b, sem_b)
      chunk_compute(t, rows_b)
      tn = t + 2
      for j in range(SUBG):
        pltpu.async_copy(emb.at[idx_v.at[tn * SUBG + j]],
                         rows_b.at[pl.ds(j * RPS, RPS)], sem_b)
    return carry

  lax.fori_loop(0, (NCHUNK - 2) // 2, step, 0)

  for t in (NCHUNK - 2, NCHUNK - 1):
    rows_b, sem_b = (rows0, sem0) if t % 2 == 0 else (rows1, sem1)
    drain(t, rows_b, sem_b)
    chunk_compute(t, rows_b)

  pltpu.sync_copy(out_v, out.at[pl.ds(wid * NCHUNK, NCHUNK)])


_FM_CALL = None


def _get_fm_call():
  global _FM_CALL
  if _FM_CALL is None:
    mesh = plsc.VectorSubcoreMesh(core_axis_name="c", subcore_axis_name="s",
                                  num_cores=NCORES, num_subcores=NSUB)
    scratch = [
        pltpu.VMEM((NWORK, RPS), jnp.int32),
        pltpu.VMEM((NCAT, EPW), jnp.int32),
        pltpu.VMEM((NCVL, EPW), jnp.float32),
        pltpu.VMEM((NCAT, EPW), jnp.float32),
        pltpu.VMEM((CHUNK * NCAT, EMB_D), jnp.float32),
        pltpu.VMEM((CHUNK * NCAT, EMB_D), jnp.float32),
        pltpu.VMEM((NCHUNK, CHUNK), jnp.float32),
        pltpu.VMEM((LANES,), jnp.float32),
        pltpu.SemaphoreType.DMA,
        pltpu.SemaphoreType.DMA,
        pltpu.SemaphoreType.DMA,
    ]
    _FM_CALL = pl.kernel(
        _fm_body,
        out_type=jax.ShapeDtypeStruct((NWORK * NCHUNK, CHUNK), jnp.float32),
        mesh=mesh,
        scratch_types=scratch,
        compiler_params=pltpu.CompilerParams(needs_layout_passes=False,
                                             use_tc_tiling_on_sc=False),
    )
  return _FM_CALL


def kernel(x, emb_weight, fc_weight, bias):
  idx = x[:, :NCAT].astype(jnp.int32)
  idx_g = idx.reshape(BATCH // EPS, EPS * NCAT)
  idx_f = jnp.transpose(idx.reshape(NWORK, EPW, NCAT), (0, 2, 1))
  cont_t = jnp.transpose(x[:, NCAT:].reshape(NWORK, EPW, NCVL), (0, 2, 1))
  fc_flat = fc_weight.reshape(-1)
  bias16 = jnp.broadcast_to(bias.astype(jnp.float32), (LANES,))
  out2 = _get_fm_call()(idx_g, idx_f, cont_t, emb_weight, fc_flat, bias16)
  return out2.reshape(BATCH)


# speedup vs baseline: 1.6286x; 1.0636x over previous
import jax
import jax.numpy as jnp
from jax import lax
from jax.experimental import pallas as pl
from jax.experimental.pallas import tpu as pltpu
from jax.experimental.pallas import tpu_sc as plsc

BATCH = 4096
NFEAT = 39
NCAT = 26
NCVL = 13
EMB_D = 64
LANES = 16

NCORES = 2
NSUB = 16
NWORK = NCORES * NSUB
EPW = BATCH // NWORK
CHUNK = 16
NCHUNK = EPW // CHUNK
SUBG = 4
EPS = CHUNK // SUBG
RPS = EPS * NCAT
IPW = EPW * NCAT
XPW = EPW * NFEAT
NGRP = IPW // LANES


def _fm_body(x, emb, fc, bias, out,
             xv, idx_v, fcv, rows0, rows1, out_v, bias_v,
             sem_fc, sem0, sem1):
  wid = lax.axis_index("s") * NCORES + lax.axis_index("c")
  lane = lax.broadcasted_iota(jnp.int32, (LANES,), 0)

  pltpu.sync_copy(x.at[pl.ds(wid * XPW, XPW)], xv)
  pltpu.sync_copy(bias, bias_v)

  def build(k, carry):
    src, fpos = carry
    v = plsc.load_gather(xv, [src])
    idx_v[pl.ds(k * LANES, LANES)] = v.astype(jnp.int32)
    fnext = fpos + LANES
    wrap = fnext >= NCAT
    fnext = jnp.where(wrap, fnext - NCAT, fnext)
    src = src + LANES + jnp.where(wrap, NFEAT - NCAT, 0)
    return src, fnext

  lax.fori_loop(0, NGRP, build, (lane, lane))

  fc_descs = [
      pltpu.async_copy(fc.at[idx_v.at[pl.ds(g * RPS, RPS)]],
                       fcv.at[pl.ds(g * RPS, RPS)], sem_fc)
      for g in range(IPW // RPS)
  ]

  for t in range(2):
    rows_b, sem_b = (rows0, sem0) if t == 0 else (rows1, sem1)
    for j in range(SUBG):
      pltpu.async_copy(emb.at[idx_v.at[pl.ds(t * CHUNK * NCAT + j * RPS, RPS)]],
                       rows_b.at[pl.ds(j * RPS, RPS)], sem_b)

  for d in fc_descs:
    d.wait()
  bvec = bias_v[...]
  lane26 = lane * NCAT
  lane39 = lane * NFEAT
  for c in range(NCHUNK):
    acc = bvec
    for f in range(NCAT):
      v = plsc.load_gather(fcv, [lane26 + (c * CHUNK * NCAT + f)])
      if f >= NCAT - NCVL:
        cv = plsc.load_gather(
            xv, [lane39 + (c * CHUNK * NFEAT + NCAT + f - (NCAT - NCVL))])
        v = v * cv
      acc = acc + v
    out_v[c, :] = acc

  def chunk_compute(t, rows_b):
    def elem(i, acc):
      base = i * NCAT
      zero = jnp.zeros((LANES,), jnp.float32)
      s0 = s1 = s2 = s3 = q = zero
      for f in range(NCAT):
        r = base + f
        e0 = rows_b[r, pl.ds(0, 16)]
        e1 = rows_b[r, pl.ds(16, 16)]
        e2 = rows_b[r, pl.ds(32, 16)]
        e3 = rows_b[r, pl.ds(48, 16)]
        s0 = s0 + e0
        s1 = s1 + e1
        s2 = s2 + e2
        s3 = s3 + e3
        q = q + (e0 * e0 + e1 * e1 + e2 * e2 + e3 * e3)
      ssq = s0 * s0 + s1 * s1 + s2 * s2 + s3 * s3
      val = 0.5 * (jnp.sum(ssq) - jnp.sum(q))
      return acc + jnp.where(lane == i, val, 0.0)

    inter = lax.fori_loop(0, CHUNK, elem, jnp.zeros((LANES,), jnp.float32))
    z = out_v[t, :] + inter
    out_v[t, :] = 1.0 / (1.0 + jnp.exp(-z))

  def drain(t, rows_b, sem_b):
    for j in range(SUBG):
      pltpu.make_async_copy(
          emb.at[idx_v.at[pl.ds(t * CHUNK * NCAT + j * RPS, RPS)]],
          rows_b.at[pl.ds(j * RPS, RPS)], sem_b).wait()

  def step(g, carry):
    for b in range(2):
      t = g * 2 + b
      rows_b, sem_b = (rows0, sem0) if b == 0 else (rows1, sem1)
      drain(t, rows_b, sem_b)
      chunk_compute(t, rows_b)
      tn = t + 2
      for j in range(SUBG):
        pltpu.async_copy(
            emb.at[idx_v.at[pl.ds(tn * CHUNK * NCAT + j * RPS, RPS)]],
            rows_b.at[pl.ds(j * RPS, RPS)], sem_b)
    return carry

  lax.fori_loop(0, (NCHUNK - 2) // 2, step, 0)

  for t in (NCHUNK - 2, NCHUNK - 1):
    rows_b, sem_b = (rows0, sem0) if t % 2 == 0 else (rows1, sem1)
    drain(t, rows_b, sem_b)
    chunk_compute(t, rows_b)

  pltpu.sync_copy(out_v, out.at[pl.ds(wid * NCHUNK, NCHUNK)])


_FM_CALL = None


def _get_fm_call():
  global _FM_CALL
  if _FM_CALL is None:
    mesh = plsc.VectorSubcoreMesh(core_axis_name="c", subcore_axis_name="s",
                                  num_cores=NCORES, num_subcores=NSUB)
    scratch = [
        pltpu.VMEM((XPW,), jnp.float32),
        pltpu.VMEM((IPW,), jnp.int32),
        pltpu.VMEM((IPW,), jnp.float32),
        pltpu.VMEM((CHUNK * NCAT, EMB_D), jnp.float32),
        pltpu.VMEM((CHUNK * NCAT, EMB_D), jnp.float32),
        pltpu.VMEM((NCHUNK, CHUNK), jnp.float32),
        pltpu.VMEM((LANES,), jnp.float32),
        pltpu.SemaphoreType.DMA,
        pltpu.SemaphoreType.DMA,
        pltpu.SemaphoreType.DMA,
    ]
    _FM_CALL = pl.kernel(
        _fm_body,
        out_type=jax.ShapeDtypeStruct((NWORK * NCHUNK, CHUNK), jnp.float32),
        mesh=mesh,
        scratch_types=scratch,
        compiler_params=pltpu.CompilerParams(needs_layout_passes=False,
                                             use_tc_tiling_on_sc=False),
    )
  return _FM_CALL


def kernel(x, emb_weight, fc_weight, bias):
  x_flat = x.reshape(-1)
  fc_flat = fc_weight.reshape(-1)
  bias16 = jnp.broadcast_to(bias.astype(jnp.float32), (LANES,))
  out2 = _get_fm_call()(x_flat, emb_weight, fc_flat, bias16)
  return out2.reshape(BATCH)


# speedup vs baseline: 1.8497x; 1.1357x over previous
import jax
import jax.numpy as jnp
from jax import lax
from jax.experimental import pallas as pl
from jax.experimental.pallas import tpu as pltpu
from jax.experimental.pallas import tpu_sc as plsc

BATCH = 4096
NFEAT = 39
NCAT = 26
NCVL = 13
EMB_D = 64
LANES = 16

NCORES = 2
NSUB = 16
NWORK = NCORES * NSUB
EPW = BATCH // NWORK
CHUNK = 16
NCHUNK = EPW // CHUNK
SUBG = 4
EPS = CHUNK // SUBG
RPS = EPS * NCAT
ROW_W = EMB_D
IPW = EPW * NCAT
XPW = EPW * NFEAT
NGRP = IPW // LANES


def _fm_body(x, emb, fc, bias, out,
             xv, idx_v, fcv, rows0, rows1, out_v, bias_v,
             sem_fc, sem0, sem1):
  wid = lax.axis_index("s") * NCORES + lax.axis_index("c")
  lane = lax.broadcasted_iota(jnp.int32, (LANES,), 0)

  pltpu.sync_copy(x.at[pl.ds(wid * XPW, XPW)], xv)
  pltpu.sync_copy(bias, bias_v)

  def build(k, carry):
    src, fpos = carry
    v = plsc.load_gather(xv, [src])
    idx_v[pl.ds(k * LANES, LANES)] = v.astype(jnp.int32)
    fnext = fpos + LANES
    wrap = fnext >= NCAT
    fnext = jnp.where(wrap, fnext - NCAT, fnext)
    src = src + LANES + jnp.where(wrap, NFEAT - NCAT, 0)
    return src, fnext

  lax.fori_loop(0, NGRP, build, (lane, lane))

  fc_descs = [
      pltpu.async_copy(fc.at[idx_v.at[pl.ds(g * RPS, RPS)]],
                       fcv.at[pl.ds(g * RPS, RPS)], sem_fc)
      for g in range(IPW // RPS)
  ]

  for t in range(2):
    rows_b, sem_b = (rows0, sem0) if t == 0 else (rows1, sem1)
    for j in range(SUBG):
      pltpu.async_copy(emb.at[idx_v.at[pl.ds(t * CHUNK * NCAT + j * RPS, RPS)]],
                       rows_b.at[pl.ds(j * RPS, RPS)], sem_b)

  for d in fc_descs:
    d.wait()
  bvec = bias_v[...]
  lane26 = lane * NCAT
  lane39 = lane * NFEAT
  for c in range(NCHUNK):
    acc = bvec
    for f in range(NCAT):
      v = plsc.load_gather(fcv, [lane26 + (c * CHUNK * NCAT + f)])
      if f >= NCAT - NCVL:
        cv = plsc.load_gather(
            xv, [lane39 + (c * CHUNK * NFEAT + NCAT + f - (NCAT - NCVL))])
        v = v * cv
      acc = acc + v
    out_v[pl.ds(c * CHUNK, CHUNK)] = acc

  def chunk_compute(t, rows_b):
    def elem(i, acc):
      base = i * NCAT
      zero = jnp.zeros((LANES,), jnp.float32)
      s0 = s1 = s2 = s3 = zero
      q0 = q1 = q2 = q3 = zero
      for f in range(NCAT):
        r = base + f
        e0 = rows_b[r, pl.ds(0, 16)]
        e1 = rows_b[r, pl.ds(16, 16)]
        e2 = rows_b[r, pl.ds(32, 16)]
        e3 = rows_b[r, pl.ds(48, 16)]
        s0 = s0 + e0
        s1 = s1 + e1
        s2 = s2 + e2
        s3 = s3 + e3
        q0 = q0 + e0 * e0
        q1 = q1 + e1 * e1
        q2 = q2 + e2 * e2
        q3 = q3 + e3 * e3
      d = (s0 * s0 - q0) + (s1 * s1 - q1) + (s2 * s2 - q2) + (s3 * s3 - q3)
      val = 0.5 * jnp.sum(d)
      return acc + jnp.where(lane == i, val, 0.0)

    off = pl.multiple_of(t * CHUNK, CHUNK)
    inter = lax.fori_loop(0, CHUNK, elem, jnp.zeros((LANES,), jnp.float32))
    z = out_v[pl.ds(off, CHUNK)] + inter
    out_v[pl.ds(off, CHUNK)] = 1.0 / (1.0 + jnp.exp(-z))

  def drain(t, rows_b, sem_b):
    for j in range(SUBG):
      pltpu.make_async_copy(
          emb.at[idx_v.at[pl.ds(t * CHUNK * NCAT + j * RPS, RPS)]],
          rows_b.at[pl.ds(j * RPS, RPS)], sem_b).wait()

  def step(g, carry):
    for b in range(2):
      t = g * 2 + b
      rows_b, sem_b = (rows0, sem0) if b == 0 else (rows1, sem1)
      drain(t, rows_b, sem_b)
      chunk_compute(t, rows_b)
      tn = t + 2
      for j in range(SUBG):
        pltpu.async_copy(
            emb.at[idx_v.at[pl.ds(tn * CHUNK * NCAT + j * RPS, RPS)]],
            rows_b.at[pl.ds(j * RPS, RPS)], sem_b)
    return carry

  lax.fori_loop(0, (NCHUNK - 2) // 2, step, 0)

  for t in (NCHUNK - 2, NCHUNK - 1):
    rows_b, sem_b = (rows0, sem0) if t % 2 == 0 else (rows1, sem1)
    drain(t, rows_b, sem_b)
    chunk_compute(t, rows_b)

  pltpu.sync_copy(out_v, out.at[pl.ds(wid * EPW, EPW)])


_FM_CALL = None


def _get_fm_call():
  global _FM_CALL
  if _FM_CALL is None:
    mesh = plsc.VectorSubcoreMesh(core_axis_name="c", subcore_axis_name="s",
                                  num_cores=NCORES, num_subcores=NSUB)
    scratch = [
        pltpu.VMEM((XPW,), jnp.float32),
        pltpu.VMEM((IPW,), jnp.int32),
        pltpu.VMEM((IPW,), jnp.float32),
        pltpu.VMEM((CHUNK * NCAT, ROW_W), jnp.float32),
        pltpu.VMEM((CHUNK * NCAT, ROW_W), jnp.float32),
        pltpu.VMEM((EPW,), jnp.float32),
        pltpu.VMEM((LANES,), jnp.float32),
        pltpu.SemaphoreType.DMA,
        pltpu.SemaphoreType.DMA,
        pltpu.SemaphoreType.DMA,
    ]
    _FM_CALL = pl.kernel(
        _fm_body,
        out_type=jax.ShapeDtypeStruct((BATCH,), jnp.float32),
        mesh=mesh,
        scratch_types=scratch,
        compiler_params=pltpu.CompilerParams(needs_layout_passes=False,
                                             use_tc_tiling_on_sc=False),
    )
  return _FM_CALL


def kernel(x, emb_weight, fc_weight, bias):
  x_flat = x.reshape(-1)
  fc_flat = fc_weight.reshape(-1)
  bias16 = jnp.broadcast_to(bias.astype(jnp.float32), (LANES,))
  return _get_fm_call()(x_flat, emb_weight, fc_flat, bias16)
